# baseline (speedup 1.0000x reference)
import functools

import jax
import jax.numpy as jnp
from jax import lax
from jax.experimental import pallas as pl
from jax.experimental.pallas import tpu as pltpu

_B, _S, _D = 2, 2048, 2048
_E, _K, _H = 64, 2, 4
_DL, _DH = 64, 128
_NSUB, _DHALF = 8, 64
_N = _B * _S
_BLK = 1024
_NBLK = _N // _BLK
_QW = 2 * _H * _DHALF
_SW = 2 * _H * _NSUB

_NEG_INF = float("-inf")


def _top2_ax1(s, width):
    iota = lax.broadcasted_iota(jnp.int32, s.shape, 1)
    v0 = jnp.max(s, axis=1, keepdims=True)
    i0 = jnp.min(jnp.where(s >= v0, iota, width), axis=1, keepdims=True)
    sm = jnp.where(iota == i0, _NEG_INF, s)
    v1 = jnp.max(sm, axis=1, keepdims=True)
    i1 = jnp.min(jnp.where(sm >= v1, iota, width), axis=1, keepdims=True)
    return v0, i0, v1, i1


def _moe_body(x_ref, wqr_ref, wu_ref, skbig_ref, el_ref, w1_ref, wv_ref,
              o_ref, h_all_ref, wd_ref):
    @pl.when(pl.program_id(0) == 0)
    def _init():
        h = jnp.dot(el_ref[...], w1_ref[...], preferred_element_type=jnp.float32)
        h = h * jax.nn.sigmoid(1.702 * h)
        h_all_ref[...] = h
        wd_ref[...] = lax.dot_general(
            wu_ref[...], h, (((1,), (1,)), ((), ())),
            preferred_element_type=jnp.float32)

    h_all = h_all_ref[...]
    xb = x_ref[...]
    q = jnp.dot(xb, wqr_ref[...], preferred_element_type=jnp.float32)
    dot_all = jnp.dot(xb, wd_ref[...], preferred_element_type=jnp.float32)
    dot_all_t = jnp.transpose(dot_all)

    s_all = jnp.dot(q, skbig_ref[...], preferred_element_type=jnp.float32)
    st = jnp.transpose(s_all)
    s1t = st[:_H * _NSUB].reshape(_H, _NSUB, _BLK)
    s2t = st[_H * _NSUB:].reshape(_H, _NSUB, _BLK)

    v1a, i1a, v1b, i1b = _top2_ax1(s1t, _NSUB)
    v2a, i2a, v2b, i2b = _top2_ax1(s2t, _NSUB)
    comb = jnp.concatenate(
        [v1a + v2a, v1a + v2b, v1b + v2a, v1b + v2b], axis=1)
    cidx = jnp.concatenate(
        [i1a * _NSUB + i2a, i1a * _NSUB + i2b,
         i1b * _NSUB + i2a, i1b * _NSUB + i2b], axis=1)
    sc0, p0, sc1, p1 = _top2_ax1(comb, _K * _K)
    iota4 = lax.broadcasted_iota(jnp.int32, (_H, _K * _K, _BLK), 1)
    e0 = jnp.sum(jnp.where(iota4 == p0, cidx, 0), axis=1, keepdims=True)
    e1 = jnp.sum(jnp.where(iota4 == p1, cidx, 0), axis=1, keepdims=True)
    ex = jnp.exp(sc1 - sc0)
    denom = 1.0 + ex
    sw0 = 1.0 / denom
    sw1 = ex / denom

    iota_e = lax.broadcasted_iota(jnp.int32, (_E, _BLK), 0)
    w_acc = jnp.zeros((_E, _BLK), jnp.float32)
    for h in range(_H):
        m0 = iota_e == e0[h]
        m1 = iota_e == e1[h]
        d0 = jnp.sum(jnp.where(m0, dot_all_t, 0.0), axis=0, keepdims=True)
        d1 = jnp.sum(jnp.where(m1, dot_all_t, 0.0), axis=0, keepdims=True)
        act0 = d0 * jax.nn.sigmoid(1.702 * d0) * sw0[h]
        act1 = d1 * jax.nn.sigmoid(1.702 * d1) * sw1[h]
        w_acc = w_acc + jnp.where(m0, act0, 0.0) + jnp.where(m1, act1, 0.0)

    c = jnp.dot(jnp.transpose(w_acc), h_all,
                preferred_element_type=jnp.float32)
    o_ref[...] = jnp.dot(c, wv_ref[...],
                         preferred_element_type=jnp.float32) * (1.0 / _H)


@functools.partial(jax.jit, static_argnames=())
def _run(xf, wqr, wu, skbig, el, w1, wv):
    return pl.pallas_call(
        _moe_body,
        grid=(_NBLK,),
        in_specs=[
            pl.BlockSpec((_BLK, _D), lambda i: (i, 0)),
            pl.BlockSpec((_D, _QW), lambda i: (0, 0)),
            pl.BlockSpec((_D, _DH), lambda i: (0, 0)),
            pl.BlockSpec((_QW, _SW), lambda i: (0, 0)),
            pl.BlockSpec((_E, _DL), lambda i: (0, 0)),
            pl.BlockSpec((_DL, _DH), lambda i: (0, 0)),
            pl.BlockSpec((_DH, _D), lambda i: (0, 0)),
        ],
        out_specs=pl.BlockSpec((_BLK, _D), lambda i: (i, 0)),
        out_shape=jax.ShapeDtypeStruct((_N, _D), jnp.float32),
        scratch_shapes=[
            pltpu.VMEM((_E, _DH), jnp.float32),
            pltpu.VMEM((_D, _E), jnp.float32),
        ],
        compiler_params=pltpu.CompilerParams(
            dimension_semantics=("arbitrary",)),
    )(xf, wqr, wu, skbig, el, w1, wv)


def kernel(x, expert_latents, W1, W2, Wq, sub_keys):
    xf = x.reshape(_N, _D)
    wqr = Wq.reshape(_D, _H, 2, _DHALF).transpose(0, 2, 1, 3).reshape(_D, _QW)
    wu = W2[:, :_D].T
    skbig = jax.scipy.linalg.block_diag(
        *([sub_keys[0, h].T for h in range(_H)]
          + [sub_keys[1, h].T for h in range(_H)]))
    wv = W2[:, _D:]
    out = _run(xf, wqr, wu, skbig, expert_latents, W1, wv)
    return out.reshape(_B, _S, _D)


# speedup vs baseline: 1.1308x; 1.1308x over previous
import functools

import jax
import jax.numpy as jnp
from jax import lax
from jax.experimental import pallas as pl
from jax.experimental.pallas import tpu as pltpu

_B, _S, _D = 2, 2048, 2048
_E, _K, _H = 64, 2, 4
_DL, _DH = 64, 128
_NSUB, _DHALF = 8, 64
_N = _B * _S
_BLK = 1024
_NBLK = _N // _BLK
_QW = 2 * _H * _DHALF
_SW = 2 * _H * _NSUB

_NEG_INF = float("-inf")


def _top2_ax1(s, width):
    iota = lax.broadcasted_iota(jnp.int32, s.shape, 1)
    v0 = jnp.max(s, axis=1, keepdims=True)
    i0 = jnp.min(jnp.where(s >= v0, iota, width), axis=1, keepdims=True)
    sm = jnp.where(iota == i0, _NEG_INF, s)
    v1 = jnp.max(sm, axis=1, keepdims=True)
    i1 = jnp.min(jnp.where(sm >= v1, iota, width), axis=1, keepdims=True)
    return v0, i0, v1, i1


def _moe_body(x_ref, wqr_ref, wu_ref, skbig_ref, el_ref, w1_ref, wv_ref,
              o_ref, h_all_ref, wbig_ref):
    @pl.when(pl.program_id(0) == 0)
    def _init():
        h = jnp.dot(el_ref[...], w1_ref[...], preferred_element_type=jnp.float32)
        h = h * jax.nn.sigmoid(1.702 * h)
        h_all_ref[...] = h
        wbig_ref[:, :_SW] = jnp.dot(wqr_ref[...], skbig_ref[...],
                                    preferred_element_type=jnp.float32)
        wbig_ref[:, _SW:] = lax.dot_general(
            wu_ref[...], h, (((1,), (1,)), ((), ())),
            preferred_element_type=jnp.float32)

    h_all = h_all_ref[...]
    xb = x_ref[...]
    sd = jnp.dot(xb, wbig_ref[...], preferred_element_type=jnp.float32)
    sdt = jnp.transpose(sd)
    dot_all_t = sdt[_SW:]
    st = sdt[:_SW]
    s1t = st[:_H * _NSUB].reshape(_H, _NSUB, _BLK)
    s2t = st[_H * _NSUB:].reshape(_H, _NSUB, _BLK)

    v1a, i1a, v1b, i1b = _top2_ax1(s1t, _NSUB)
    v2a, i2a, v2b, i2b = _top2_ax1(s2t, _NSUB)
    comb = jnp.concatenate(
        [v1a + v2a, v1a + v2b, v1b + v2a, v1b + v2b], axis=1)
    cidx = jnp.concatenate(
        [i1a * _NSUB + i2a, i1a * _NSUB + i2b,
         i1b * _NSUB + i2a, i1b * _NSUB + i2b], axis=1)
    sc0, p0, sc1, p1 = _top2_ax1(comb, _K * _K)
    iota4 = lax.broadcasted_iota(jnp.int32, (_H, _K * _K, _BLK), 1)
    e0 = jnp.sum(jnp.where(iota4 == p0, cidx, 0), axis=1, keepdims=True)
    e1 = jnp.sum(jnp.where(iota4 == p1, cidx, 0), axis=1, keepdims=True)
    ex = jnp.exp(sc1 - sc0)
    denom = 1.0 + ex
    sw0 = 1.0 / denom
    sw1 = ex / denom

    iota_e = lax.broadcasted_iota(jnp.int32, (_E, _BLK), 0)
    w_acc = jnp.zeros((_E, _BLK), jnp.float32)
    for h in range(_H):
        m0 = iota_e == e0[h]
        m1 = iota_e == e1[h]
        d0 = jnp.sum(jnp.where(m0, dot_all_t, 0.0), axis=0, keepdims=True)
        d1 = jnp.sum(jnp.where(m1, dot_all_t, 0.0), axis=0, keepdims=True)
        act0 = d0 * jax.nn.sigmoid(1.702 * d0) * sw0[h]
        act1 = d1 * jax.nn.sigmoid(1.702 * d1) * sw1[h]
        w_acc = w_acc + jnp.where(m0, act0, 0.0) + jnp.where(m1, act1, 0.0)

    c = jnp.dot(jnp.transpose(w_acc), h_all,
                preferred_element_type=jnp.float32)
    o_ref[...] = jnp.dot(c, wv_ref[...],
                         preferred_element_type=jnp.float32) * (1.0 / _H)


@functools.partial(jax.jit, static_argnames=())
def _run(xf, wqr, wu, skbig, el, w1, wv):
    return pl.pallas_call(
        _moe_body,
        grid=(_NBLK,),
        in_specs=[
            pl.BlockSpec((_BLK, _D), lambda i: (i, 0)),
            pl.BlockSpec((_D, _QW), lambda i: (0, 0)),
            pl.BlockSpec((_D, _DH), lambda i: (0, 0)),
            pl.BlockSpec((_QW, _SW), lambda i: (0, 0)),
            pl.BlockSpec((_E, _DL), lambda i: (0, 0)),
            pl.BlockSpec((_DL, _DH), lambda i: (0, 0)),
            pl.BlockSpec((_DH, _D), lambda i: (0, 0)),
        ],
        out_specs=pl.BlockSpec((_BLK, _D), lambda i: (i, 0)),
        out_shape=jax.ShapeDtypeStruct((_N, _D), jnp.float32),
        scratch_shapes=[
            pltpu.VMEM((_E, _DH), jnp.float32),
            pltpu.VMEM((_D, _SW + _E), jnp.float32),
        ],
        compiler_params=pltpu.CompilerParams(
            dimension_semantics=("arbitrary",)),
    )(xf, wqr, wu, skbig, el, w1, wv)


def kernel(x, expert_latents, W1, W2, Wq, sub_keys):
    xf = x.reshape(_N, _D)
    wqr = Wq.reshape(_D, _H, 2, _DHALF).transpose(0, 2, 1, 3).reshape(_D, _QW)
    wu = W2[:, :_D].T
    skbig = jax.scipy.linalg.block_diag(
        *([sub_keys[0, h].T for h in range(_H)]
          + [sub_keys[1, h].T for h in range(_H)]))
    wv = W2[:, _D:]
    out = _run(xf, wqr, wu, skbig, expert_latents, W1, wv)
    return out.reshape(_B, _S, _D)
